# Initial kernel scaffold; baseline (speedup 1.0000x reference)
import jax
import jax.numpy as jnp
from jax.experimental import pallas as pl


def kernel(input, table, W, b):
    raise NotImplementedError("write your pallas kernel here")


# speedup vs baseline: 5.1775x; 5.1775x over previous
import functools

import jax
import jax.numpy as jnp
from jax import lax
from jax.experimental import pallas as pl
from jax.experimental.pallas import tpu as pltpu
from jax.experimental.pallas import tpu_sc as plsc

BATCH = 4096
MAX_LEN = 200
WORD_DIM = 64
N_LABELS = 100
TOKENS = BATCH * MAX_LEN
GCHUNK = 128
NW = 32
ROWS_PER_W = TOKENS // NW
CHUNKS_PER_W = ROWS_PER_W // GCHUNK


def _sc_gather(table, idx2d):
    mesh = plsc.VectorSubcoreMesh(core_axis_name="c", subcore_axis_name="s")

    @functools.partial(
        pl.kernel,
        out_type=jax.ShapeDtypeStruct((TOKENS, WORD_DIM), jnp.float32),
        mesh=mesh,
        compiler_params=pltpu.CompilerParams(use_tc_tiling_on_sc=False),
        scratch_types=[
            pltpu.VMEM((CHUNKS_PER_W, GCHUNK), jnp.int32),
            pltpu.VMEM((2, GCHUNK, WORD_DIM), jnp.float32),
            pltpu.SemaphoreType.DMA,
            pltpu.SemaphoreType.DMA,
        ],
    )
    def k(table_hbm, idx_hbm, out_hbm, idx_v, rows_v, gsem, osem):
        wid = lax.axis_index("s") * 2 + lax.axis_index("c")
        base_chunk = wid * CHUNKS_PER_W
        pltpu.sync_copy(idx_hbm.at[pl.ds(base_chunk, CHUNKS_PER_W)], idx_v)

        def gather_start(j, slot):
            return pltpu.async_copy(table_hbm.at[idx_v.at[j]], rows_v.at[slot],
                                    gsem)

        def out_start(j, slot):
            return pltpu.async_copy(
                rows_v.at[slot],
                out_hbm.at[pl.ds((base_chunk + j) * GCHUNK, GCHUNK)],
                osem)

        gather_start(0, 0).wait()

        def body(j, _):
            slot = lax.rem(j, 2)

            @pl.when(j + 1 < CHUNKS_PER_W)
            def _():
                gather_start(j + 1, 1 - slot).wait()

            out_start(j, slot).wait()
            return 0

        lax.fori_loop(0, CHUNKS_PER_W, body, 0)

    return k(table, idx2d)


def _tc_matmul(flat, Wt, b2d):
    BB = 256
    K = MAX_LEN * WORD_DIM

    def mm(x_ref, w_ref, b_ref, o_ref):
        o_ref[...] = (
            jnp.dot(x_ref[...], w_ref[...], preferred_element_type=jnp.float32)
            + b_ref[...])

    return pl.pallas_call(
        mm,
        grid=(BATCH // BB,),
        in_specs=[
            pl.BlockSpec((BB, K), lambda i: (i, 0)),
            pl.BlockSpec((K, N_LABELS), lambda i: (0, 0)),
            pl.BlockSpec((1, N_LABELS), lambda i: (0, 0)),
        ],
        out_specs=pl.BlockSpec((BB, N_LABELS), lambda i: (i, 0)),
        out_shape=jax.ShapeDtypeStruct((BATCH, N_LABELS), jnp.float32),
    )(flat, Wt, b2d)


def kernel(input, table, W, b):
    idx2d = input.reshape(TOKENS // GCHUNK, GCHUNK).astype(jnp.int32)
    rows = _sc_gather(table, idx2d)
    flat = rows.reshape(BATCH, MAX_LEN * WORD_DIM)
    return _tc_matmul(flat, W.T, b.reshape(1, N_LABELS))


# speedup vs baseline: 9.2743x; 1.7913x over previous
import functools

import jax
import jax.numpy as jnp
from jax import lax
from jax.experimental import pallas as pl
from jax.experimental.pallas import tpu as pltpu
from jax.experimental.pallas import tpu_sc as plsc

BATCH = 4096
MAX_LEN = 200
WORD_DIM = 64
N_LABELS = 100
RB = MAX_LEN // 2
OUT_ROWS = BATCH * RB
NW = 32
B_PER_W = BATCH // NW
G1 = 104
G2 = MAX_LEN - G1


def _sc_gather(table, idx):
    mesh = plsc.VectorSubcoreMesh(core_axis_name="c", subcore_axis_name="s")

    @functools.partial(
        pl.kernel,
        out_type=jax.ShapeDtypeStruct((OUT_ROWS, 128), jnp.float32),
        mesh=mesh,
        compiler_params=pltpu.CompilerParams(use_tc_tiling_on_sc=False),
        scratch_types=[
            pltpu.VMEM((B_PER_W, MAX_LEN), jnp.int32),
            pltpu.VMEM((2, MAX_LEN, WORD_DIM), jnp.float32),
            pltpu.VMEM((2, RB, 128), jnp.float32),
            pltpu.SemaphoreType.DMA,
            pltpu.SemaphoreType.DMA,
        ],
    )
    def k(table_hbm, idx_hbm, out_hbm, idx_v, ga_v, gb_v, gsem, osem):
        wid = lax.axis_index("s") * 2 + lax.axis_index("c")
        b0 = wid * B_PER_W
        pltpu.sync_copy(idx_hbm.at[pl.ds(b0, B_PER_W)], idx_v)

        def gather_pair(i, slot):
            return (
                pltpu.make_async_copy(
                    table_hbm.at[idx_v.at[i, pl.ds(0, G1)]],
                    ga_v.at[slot, pl.ds(0, G1)], gsem),
                pltpu.make_async_copy(
                    table_hbm.at[idx_v.at[i, pl.ds(G1, G2)]],
                    ga_v.at[slot, pl.ds(G1, G2)], gsem),
            )

        def out_copy(i, slot):
            return pltpu.make_async_copy(
                gb_v.at[slot],
                out_hbm.at[pl.ds((b0 + i) * RB, RB)], osem)

        for c in gather_pair(0, 0):
            c.start()

        def body(i, _):
            slot = lax.rem(i, 2)
            for c in gather_pair(i, slot):
                c.wait()

            @pl.when(i + 1 < B_PER_W)
            def _():
                for c in gather_pair(i + 1, 1 - slot):
                    c.start()

            @pl.when(i >= 2)
            def _():
                out_copy(i - 2, slot).wait()

            for w in range(0, MAX_LEN * WORD_DIM, 16):
                gb_v[slot, w // 128, pl.ds(w % 128, 16)] = (
                    ga_v[slot, w // WORD_DIM, pl.ds(w % WORD_DIM, 16)])

            out_copy(i, slot).start()
            return 0

        lax.fori_loop(0, B_PER_W, body, 0)
        out_copy(B_PER_W - 2, 0).wait()
        out_copy(B_PER_W - 1, 1).wait()

    return k(table, idx)


def _tc_matmul(x, Wr, b2d):
    BB = 256

    def mm(x_ref, w_ref, b_ref, o_ref):
        x3 = x_ref[...].reshape(BB, RB, 128)
        acc = jnp.zeros((BB, N_LABELS), jnp.float32)
        for r in range(RB):
            acc += jnp.dot(x3[:, r, :], w_ref[r],
                           preferred_element_type=jnp.float32)
        o_ref[...] = acc + b_ref[...]

    return pl.pallas_call(
        mm,
        grid=(BATCH // BB,),
        in_specs=[
            pl.BlockSpec((BB * RB, 128), lambda i: (i, 0)),
            pl.BlockSpec((RB, 128, N_LABELS), lambda i: (0, 0, 0)),
            pl.BlockSpec((1, N_LABELS), lambda i: (0, 0)),
        ],
        out_specs=pl.BlockSpec((BB, N_LABELS), lambda i: (i, 0)),
        out_shape=jax.ShapeDtypeStruct((BATCH, N_LABELS), jnp.float32),
    )(x, Wr, b2d)


def kernel(input, table, W, b):
    flat = _sc_gather(table, input.astype(jnp.int32))
    Wr = (W.reshape(N_LABELS, RB, 128).transpose(1, 2, 0))
    return _tc_matmul(flat, Wr, b.reshape(1, N_LABELS))
